# baseline (speedup 1.0000x reference)
import functools

import jax
import jax.numpy as jnp
from jax import lax
from jax.experimental import pallas as pl
from jax.experimental.pallas import tpu as pltpu
from jax.experimental.pallas import tpu_sc as plsc

NC, NS = 2, 16
NW = NC * NS
CSZ = 8192


def _sc_fields(B, EMB, V, FS, FE, Ets, XiT, XvT, xi0, wb):
  NF = FE - FS
  NP = NF * EMB
  TPT = (NP + NW - 1) // NW
  CH = B // CSZ
  TOFF = max(FS, 1) - 1
  assert B % CSZ == 0

  mesh = plsc.VectorSubcoreMesh(core_axis_name="c", subcore_axis_name="s")

  @functools.partial(
      pl.kernel,
      out_type=jax.ShapeDtypeStruct((NP, B), jnp.float32),
      mesh=mesh,
      compiler_params=pltpu.CompilerParams(
          use_tc_tiling_on_sc=False, needs_layout_passes=False),
      scratch_types=[
          pltpu.VMEM((V,), jnp.float32),
          pltpu.VMEM((CSZ,), jnp.int32),
          pltpu.VMEM((CSZ,), jnp.float32),
          pltpu.VMEM((2, EMB), jnp.float32),
      ],
  )
  def sck(er, xitr, xvtr, xi0r, wbr, outr, plane, idx_v, xv_v, wb_v):
    wid = lax.axis_index("s") * NC + lax.axis_index("c")
    pltpu.sync_copy(wbr, wb_v)

    def task(k, carry):
      p = k * NW + wid

      @pl.when(p < NP)
      def _():
        i = FS + p // EMB
        e = p % EMB

        esplat = jnp.full((16,), e, jnp.int32)
        zeros = jnp.zeros((16,), jnp.int32)
        ws = plsc.load_gather(wb_v, [zeros, esplat])
        bs = plsc.load_gather(wb_v, [zeros + 1, esplat])

        @pl.when(i > 0)
        def _():
          pltpu.sync_copy(er.at[i - 1 - TOFF, e], plane)

        for cs in range(CH):
          pltpu.sync_copy(xvtr.at[i, pl.ds(cs * CSZ, CSZ)], xv_v)

          @pl.when(i == 0)
          def _():
            pltpu.sync_copy(xi0r.at[pl.ds(cs * CSZ, CSZ)], idx_v)

            @plsc.parallel_loop(0, CSZ, step=16)
            def _f0(j):
              xiv = plsc.bitcast(idx_v[pl.ds(j, 16)], jnp.float32)
              xv_v[pl.ds(j, 16)] = (xiv * ws + bs) * xv_v[pl.ds(j, 16)]

          @pl.when(i > 0)
          def _():
            pltpu.sync_copy(xitr.at[i, pl.ds(cs * CSZ, CSZ)], idx_v)

            @plsc.parallel_loop(0, CSZ, step=16)
            def _gather(j):
              idxv = idx_v[pl.ds(j, 16)]
              vals = plsc.load_gather(plane, [idxv])
              xv_v[pl.ds(j, 16)] = vals * xv_v[pl.ds(j, 16)]

          pltpu.sync_copy(xv_v, outr.at[p, pl.ds(cs * CSZ, CSZ)])

      return carry

    lax.fori_loop(0, TPT, task, 0)

  return sck(Ets, XiT, XvT, xi0, wb)


@jax.jit
def kernel(Xi, Xv, W1, b1, E1, W2, b2, E2):
  B, L, C, D = Xi.shape
  V, EMB = E1.shape[1], E1.shape[2]
  BL = B * L
  FM = (C + 1) // 2
  XiT = Xi.reshape(BL, C).astype(jnp.int32).T
  XvT = Xv.reshape(BL, C).T
  xi0 = lax.bitcast_convert_type(XiT[0].astype(jnp.float32), jnp.int32)
  outs = []
  for Et, W, bb in ((E1.transpose(0, 2, 1), W1, b1),
                    (E2.transpose(0, 2, 1), W2, b2)):
    wb = jnp.stack([W[:, 0], bb])
    oa = _sc_fields(BL, EMB, V, 0, FM, Et[:FM - 1], XiT, XvT, xi0, wb)
    ob = _sc_fields(BL, EMB, V, FM, C, Et[FM - 1:], XiT, XvT, xi0, wb)
    outs.append(jnp.concatenate([oa, ob], axis=0))
  O1, O2 = outs
  fm_first = O1.T.reshape(B, L, C * EMB)
  fm_second = O2.reshape(C, EMB, BL).transpose(2, 0, 1)
  return fm_first, fm_second


# speedup vs baseline: 1.2380x; 1.2380x over previous
import functools

import jax
import jax.numpy as jnp
from jax import lax
from jax.experimental import pallas as pl
from jax.experimental.pallas import tpu as pltpu
from jax.experimental.pallas import tpu_sc as plsc

NC, NS = 2, 16
NW = NC * NS
CSZ = 4096


def _sc_table(B, C, EMB, V, Et, XiT, XvT, wb):
  NP = C * EMB
  TPT = (NP + NW - 1) // NW
  CH = B // CSZ
  assert B % CSZ == 0 and CH % 2 == 0

  mesh = plsc.VectorSubcoreMesh(core_axis_name="c", subcore_axis_name="s")

  @functools.partial(
      pl.kernel,
      out_type=jax.ShapeDtypeStruct((NP, B), jnp.float32),
      mesh=mesh,
      compiler_params=pltpu.CompilerParams(
          use_tc_tiling_on_sc=False, needs_layout_passes=False),
      scratch_types=[
          pltpu.VMEM((V,), jnp.float32),
          pltpu.VMEM((CSZ,), jnp.int32),
          pltpu.VMEM((CSZ,), jnp.int32),
          pltpu.VMEM((CSZ,), jnp.float32),
          pltpu.VMEM((CSZ,), jnp.float32),
          pltpu.VMEM((2, EMB), jnp.float32),
          pltpu.SemaphoreType.DMA,
          pltpu.SemaphoreType.DMA,
          pltpu.SemaphoreType.DMA,
      ],
  )
  def sck(er, xitr, xvtr, wbr, outr,
          plane, idxA, idxB, xvA, xvB, wb_v, semp, sems, semo):
    wid = lax.axis_index("s") * NC + lax.axis_index("c")
    pltpu.sync_copy(wbr, wb_v)

    def task(k, carry):
      p = k * NW + wid

      @pl.when(p < NP)
      def _():
        i = p // EMB
        e = p % EMB

        esplat = jnp.full((16,), e, jnp.int32)
        zeros = jnp.zeros((16,), jnp.int32)
        ws = plsc.load_gather(wb_v, [zeros, esplat])
        bs = plsc.load_gather(wb_v, [zeros + 1, esplat])

        pcp = pltpu.async_copy(
            er.at[jnp.maximum(i - 1, 0), e], plane, semp)

        outd = [None, None]
        for cs in range(CH):
          ib, xb = (idxA, xvA) if cs % 2 == 0 else (idxB, xvB)
          if outd[cs % 2] is not None:
            outd[cs % 2].wait()
          icp = pltpu.async_copy(
              xitr.at[i, pl.ds(cs * CSZ, CSZ)], ib, sems)
          xcp = pltpu.async_copy(
              xvtr.at[i, pl.ds(cs * CSZ, CSZ)], xb, sems)
          if cs == 0:
            pcp.wait()
          icp.wait()
          xcp.wait()

          @pl.when(i == 0)
          def _():
            @plsc.parallel_loop(0, CSZ, step=16)
            def _f0(j):
              xiv = plsc.bitcast(ib[pl.ds(j, 16)], jnp.float32)
              xb[pl.ds(j, 16)] = (xiv * ws + bs) * xb[pl.ds(j, 16)]

          @pl.when(i > 0)
          def _():
            @plsc.parallel_loop(0, CSZ, step=16)
            def _gather(j):
              vals = plsc.load_gather(plane, [ib[pl.ds(j, 16)]])
              xb[pl.ds(j, 16)] = vals * xb[pl.ds(j, 16)]

          outd[cs % 2] = pltpu.async_copy(
              xb, outr.at[p, pl.ds(cs * CSZ, CSZ)], semo)
        outd[0].wait()
        outd[1].wait()

      return carry

    lax.fori_loop(0, TPT, task, 0)

  return sck(Et, XiT, XvT, wb)


@jax.jit
def kernel(Xi, Xv, W1, b1, E1, W2, b2, E2):
  B, L, C, D = Xi.shape
  V, EMB = E1.shape[1], E1.shape[2]
  BL = B * L
  XiT = Xi.reshape(BL, C).astype(jnp.int32).T
  xi0 = lax.bitcast_convert_type(XiT[0].astype(jnp.float32), jnp.int32)
  XiTb = jnp.concatenate([xi0[None, :], XiT[1:]], axis=0)
  XvT = Xv.reshape(BL, C).T
  O1 = _sc_table(BL, C, EMB, V, E1.transpose(0, 2, 1), XiTb, XvT,
                 jnp.stack([W1[:, 0], b1]))
  O2 = _sc_table(BL, C, EMB, V, E2.transpose(0, 2, 1), XiTb, XvT,
                 jnp.stack([W2[:, 0], b2]))
  fm_first = O1.T.reshape(B, L, C * EMB)
  fm_second = O2.reshape(C, EMB, BL).transpose(2, 0, 1)
  return fm_first, fm_second
